# baseline (speedup 1.0000x reference)
import jax
import jax.numpy as jnp
from jax import lax
from jax.experimental import pallas as pl
from jax.experimental.pallas import tpu as pltpu
from jax.experimental.pallas import tpu_sc as plsc

N = 10000
E = 320000
CHUNK = 128
NWORK = 32
E_PAD = NWORK * 80 * CHUNK
NJUNK = 8
SUB_ROWS = 632
LAST_ROWS = N - 15 * SUB_ROWS
BM = 2000
GRID = N // BM


def _copy_rowrange(sid, src_ref, dst_ref):
    @pl.when(sid < 15)
    def _():
        pltpu.sync_copy(src_ref.at[pl.ds(sid * SUB_ROWS, SUB_ROWS)],
                        dst_ref.at[pl.ds(sid * SUB_ROWS, SUB_ROWS)])

    @pl.when(sid == 15)
    def _():
        pltpu.sync_copy(src_ref.at[pl.ds(15 * SUB_ROWS, LAST_ROWS)],
                        dst_ref.at[pl.ds(15 * SUB_ROWS, LAST_ROWS)])


def _make_sc_agg_deep(W: int, tc_tiling: bool, ch: int = CHUNK,
                      with_deg: bool = False):
    mesh = plsc.VectorSubcoreMesh(core_axis_name="c", subcore_axis_name="s")
    NB = 4
    cpw = (E_PAD // NWORK) // ch
    assert cpw % NB == 0
    out_type = [jax.ShapeDtypeStruct((2, N, W), jnp.float32)]
    scratch = (
        [pltpu.VMEM((ch,), jnp.int32) for _ in range(NB)]
        + [pltpu.VMEM((ch,), jnp.int32) for _ in range(NB)]
        + [pltpu.VMEM((ch, W), jnp.float32) for _ in range(NB)]
        + [pltpu.VMEM_SHARED((N + NJUNK, W), jnp.float32)]
        + [pltpu.SemaphoreType.DMA] * (4 * NB)
    )
    if with_deg:
        out_type.append(jax.ShapeDtypeStruct((2, N + NJUNK), jnp.float32))
        scratch += (
            [pltpu.VMEM((ch,), jnp.float32)]
            + [pltpu.VMEM_SHARED((N + NJUNK,), jnp.float32)]
            + [pltpu.SemaphoreType.DMA] * NB
        )

    def body(z, srcm, dstm, zero2, *rest):
        if with_deg:
            zero1 = rest[0]
            rest = rest[1:]
        part = rest[0]
        if with_deg:
            degp = rest[1]
            rest = rest[2:]
        else:
            rest = rest[1:]
        srcb = rest[0:NB]
        dstb = rest[NB:2 * NB]
        rows = rest[2 * NB:3 * NB]
        acc = rest[3 * NB]
        sems = rest[3 * NB + 1:3 * NB + 1 + 4 * NB]
        isem = sems[0:NB]
        jsem = sems[NB:2 * NB]
        gsem = sems[2 * NB:3 * NB]
        ssem = sems[3 * NB:4 * NB]
        if with_deg:
            onesv = rest[3 * NB + 1 + 4 * NB]
            dacc = rest[3 * NB + 2 + 4 * NB]
            dsem = rest[3 * NB + 3 + 4 * NB:3 * NB + 3 + 5 * NB]
        cid = lax.axis_index("c")
        sid = lax.axis_index("s")
        wid = sid * 2 + cid
        ibase = wid * cpw * ch

        def sstart(g, b):
            pltpu.async_copy(srcm.at[pl.ds(ibase + g * ch, ch)],
                             srcb[b], isem[b])

        def iwait(b):
            pltpu.make_async_copy(srcm.at[pl.ds(0, ch)],
                                  srcb[b], isem[b]).wait()

        def dstart(g, b):
            pltpu.async_copy(dstm.at[pl.ds(ibase + g * ch, ch)],
                             dstb[b], jsem[b])

        def dwait(b):
            pltpu.make_async_copy(dstm.at[pl.ds(0, ch)],
                                  dstb[b], jsem[b]).wait()

        def gstart(b):
            pltpu.async_copy(z.at[srcb[b]], rows[b], gsem[b])

        def gwait(b):
            pltpu.make_async_copy(z.at[pl.ds(0, ch)], rows[b], gsem[b]).wait()

        def sc_start(b):
            pltpu.async_copy(rows[b], acc.at[dstb[b]], ssem[b], add=True)
            if with_deg:
                pltpu.async_copy(onesv, dacc.at[dstb[b]], dsem[b], add=True)

        def sc_wait(b):
            pltpu.make_async_copy(rows[b], acc.at[dstb[b]], ssem[b]).wait()
            if with_deg:
                pltpu.make_async_copy(onesv, dacc.at[dstb[b]], dsem[b]).wait()

        for b in range(NB):
            sstart(b, b)
        dstart(0, 0)
        dstart(1, 1)
        iwait(0)
        gstart(0)
        iwait(1)
        gstart(1)
        _copy_rowrange(sid, zero2, acc)
        if with_deg:
            @pl.when(sid == 0)
            def _():
                pltpu.sync_copy(zero1, dacc)
            for i in range(ch // 16):
                onesv[pl.ds(i * 16, 16)] = jnp.ones((16,), jnp.float32)
        plsc.subcore_barrier()

        def step(o, carry):
            for k in range(NB):
                g = o * NB + k
                b = k
                b2 = (k + 2) % NB
                always_x = k < 2

                def xblk(need_swait):
                    if need_swait:
                        sc_wait(b2)
                    dstart(g + 2, b2)
                    iwait(b2)
                    gstart(b2)

                if always_x:
                    @pl.when(o > 0)
                    def _():
                        xblk(True)

                    @pl.when(o == 0)
                    def _():
                        xblk(False)
                else:
                    @pl.when(o < cpw // NB - 1)
                    def _():
                        xblk(True)
                gwait(b)

                @pl.when(o < cpw // NB - 1)
                def _():
                    sstart(o * NB + k + NB, b)
                dwait(b)
                sc_start(b)
            return carry

        lax.fori_loop(0, cpw // NB, step, 0)
        for b in range(NB):
            sc_wait(b)
        plsc.subcore_barrier()
        _copy_rowrange(sid, acc, part.at[cid])
        if with_deg:
            @pl.when(sid == 0)
            def _():
                pltpu.sync_copy(dacc, degp.at[cid])

    params = None if tc_tiling else pltpu.CompilerParams(use_tc_tiling_on_sc=False)
    return pl.kernel(body, out_type=out_type, mesh=mesh, scratch_types=scratch,
                     compiler_params=params)


_sc_agg_deg128 = _make_sc_agg_deep(128, True, ch=64, with_deg=True)
_sc_agg128 = _make_sc_agg_deep(128, True, ch=64)
_sc_agg16 = _make_sc_agg_deep(16, tc_tiling=False)


def _mm_body(x_ref, w_ref, o1_ref, o2_ref):
    d = jnp.dot(x_ref[...], w_ref[...], preferred_element_type=jnp.float32)
    k = o1_ref.shape[1]
    o1_ref[...] = d[:, :k]
    o2_ref[...] = d[:, k:]


def _layer_body(pa, pb, rd_ref, s_ref, b_ref, w_ref, o1_ref, o2_ref):
    h = jnp.maximum((pa[0] + pb[0]) * rd_ref[...] + b_ref[...] + s_ref[...], 0.0)
    d = jnp.dot(h, w_ref[...], preferred_element_type=jnp.float32)
    k = o1_ref.shape[1]
    o1_ref[...] = d[:, :k]
    o2_ref[...] = d[:, k:]


def _layer1_body(pa, pb, dt_ref, s_ref, b_ref, w_ref, o1_ref, o2_ref, rd_ref):
    r = 1.0 / jnp.maximum(dt_ref[...].sum(axis=1, keepdims=True), 1.0)
    h = jnp.maximum((pa[0] + pb[0]) * r + b_ref[...] + s_ref[...], 0.0)
    d = jnp.dot(h, w_ref[...], preferred_element_type=jnp.float32)
    k = o1_ref.shape[1]
    o1_ref[...] = d[:, :k]
    o2_ref[...] = d[:, k:]
    rd_ref[...] = r


def _final_body(pa, pb, rd_ref, s_ref, b_ref, r_ref, o_ref):
    h = jax.nn.sigmoid((pa[0] + pb[0]) * rd_ref[...] + b_ref[...] + s_ref[...])
    contrib = r_ref[...][None, :, :] * h[:, None, :]
    j = lax.broadcasted_iota(jnp.int32, contrib.shape, 2)
    contrib = jnp.where(j < 13, contrib, -jnp.inf)
    o_ref[...] = jnp.max(contrib, axis=2)[:, :13]


def _row_spec(w):
    return pl.BlockSpec((BM, w), lambda i: (i, 0))


def kernel(x, edge_index, Wl0, Wr0, b0, Wl1, Wr1, b1, Wl2, Wr2, b2, R):
    f32 = jnp.float32
    npad = E_PAD - E
    pad_iota = jnp.arange(npad, dtype=jnp.int32)
    src = jnp.concatenate([edge_index[0], pad_iota % N])
    dst = jnp.concatenate([edge_index[1], N + pad_iota % NJUNK])
    zero2_128 = jnp.zeros((N, 128), f32)
    zero2_16 = jnp.zeros((N, 16), f32)
    zero1 = jnp.zeros((N + NJUNK,), f32)

    w0 = jnp.concatenate([Wl0.T, Wr0.T], axis=1)
    w1 = jnp.concatenate([Wl1.T, Wr1.T], axis=1)
    wl2p = jnp.pad(Wl2.T, ((0, 0), (0, 3)))
    wr2p = jnp.pad(Wr2.T, ((0, 0), (0, 3)))
    w2 = jnp.concatenate([wl2p, wr2p], axis=1)
    b0r = b0.reshape(1, 128)
    b1r = b1.reshape(1, 128)
    b2r = jnp.pad(b2, (0, 3)).reshape(1, 16)
    Rp = jnp.pad(R, ((0, 3), (0, 3)))

    z0, s0 = pl.pallas_call(
        _mm_body,
        grid=(GRID,),
        in_specs=[_row_spec(128), pl.BlockSpec((128, 256), lambda i: (0, 0))],
        out_specs=[_row_spec(128), _row_spec(128)],
        out_shape=[jax.ShapeDtypeStruct((N, 128), f32)] * 2,
    )(x, w0)

    part0, degp = _sc_agg_deg128(z0, src, dst, zero2_128, zero1)
    degt = degp[:, :N].T

    z1, s1, rdeg = pl.pallas_call(
        _layer1_body,
        grid=(GRID,),
        in_specs=[
            pl.BlockSpec((1, BM, 128), lambda i: (0, i, 0)),
            pl.BlockSpec((1, BM, 128), lambda i: (1, i, 0)),
            _row_spec(2),
            _row_spec(128),
            pl.BlockSpec((1, 128), lambda i: (0, 0)),
            pl.BlockSpec((128, 256), lambda i: (0, 0)),
        ],
        out_specs=[_row_spec(128), _row_spec(128), _row_spec(1)],
        out_shape=[jax.ShapeDtypeStruct((N, 128), f32),
                   jax.ShapeDtypeStruct((N, 128), f32),
                   jax.ShapeDtypeStruct((N, 1), f32)],
    )(part0, part0, degt, s0, b0r, w1)
    part1 = _sc_agg128(z1, src, dst, zero2_128)[0]

    z2, s2 = pl.pallas_call(
        _layer_body,
        grid=(GRID,),
        in_specs=[
            pl.BlockSpec((1, BM, 128), lambda i: (0, i, 0)),
            pl.BlockSpec((1, BM, 128), lambda i: (1, i, 0)),
            _row_spec(1),
            _row_spec(128),
            pl.BlockSpec((1, 128), lambda i: (0, 0)),
            pl.BlockSpec((128, 32), lambda i: (0, 0)),
        ],
        out_specs=[_row_spec(16), _row_spec(16)],
        out_shape=[jax.ShapeDtypeStruct((N, 16), f32),
                   jax.ShapeDtypeStruct((N, 16), f32)],
    )(part1, part1, rdeg, s1, b1r, w2)
    part2 = _sc_agg16(z2, src, dst, zero2_16)[0]

    out = pl.pallas_call(
        _final_body,
        grid=(GRID,),
        in_specs=[
            pl.BlockSpec((1, BM, 16), lambda i: (0, i, 0)),
            pl.BlockSpec((1, BM, 16), lambda i: (1, i, 0)),
            _row_spec(1),
            _row_spec(16),
            pl.BlockSpec((1, 16), lambda i: (0, 0)),
            pl.BlockSpec((16, 16), lambda i: (0, 0)),
        ],
        out_specs=_row_spec(13),
        out_shape=jax.ShapeDtypeStruct((N, 13), f32),
    )(part2, part2, rdeg, s2, b2r, Rp)
    return out
